# baseline (speedup 1.0000x reference)
import functools

import jax
import jax.numpy as jnp
from jax import lax
from jax.experimental import pallas as pl
from jax.experimental.pallas import tpu as pltpu
from jax.experimental.pallas import tpu_sc as plsc

_NC = 2
_NS = 16
_NW = _NC * _NS
_L = 16

_B = 16384
_D = 64
_DIM0, _DIM1, _DIM2 = 100, 100, 100
_BPW = _B // _NW
_CH = 128
_NCH = _BPW // _CH


def _make_sc_call():
    mesh = plsc.VectorSubcoreMesh(core_axis_name="c", subcore_axis_name="s")

    @functools.partial(
        pl.kernel,
        mesh=mesh,
        out_type=jax.ShapeDtypeStruct((_B, _D), jnp.float32),
        compiler_params=pltpu.CompilerParams(use_tc_tiling_on_sc=False),
        scratch_types=[
            pltpu.VMEM((_BPW,), jnp.int32),
            pltpu.VMEM((_BPW,), jnp.int32),
            pltpu.VMEM((_BPW,), jnp.int32),
            [pltpu.VMEM((_CH,), jnp.int32) for _ in range(_NCH)],
            pltpu.VMEM((_BPW, _D), jnp.float32),
            [pltpu.SemaphoreType.DMA for _ in range(_NCH)],
            pltpu.SemaphoreType.DMA,
        ],
    )
    def tabular_gather(state_hbm, params_hbm, out_hbm,
                       s0_v, s1_v, s2_v, idx_vs, rows_v, gsems, osem):
        wid = lax.axis_index("s") * _NC + lax.axis_index("c")
        base = wid * _BPW

        pltpu.sync_copy(state_hbm.at[pl.ds(0 * _B + base, _BPW)], s0_v)
        pltpu.sync_copy(state_hbm.at[pl.ds(1 * _B + base, _BPW)], s1_v)
        pltpu.sync_copy(state_hbm.at[pl.ds(2 * _B + base, _BPW)], s2_v)

        for k in range(_BPW // _L):
            sl = pl.ds(k * _L, _L)
            a = jnp.minimum(jnp.maximum(s0_v[sl], 0), _DIM0 - 1)
            b = jnp.minimum(jnp.maximum(s1_v[sl], 0), _DIM1 - 1)
            c = jnp.minimum(jnp.maximum(s2_v[sl], 0), _DIM2 - 1)
            idx = a * (_DIM1 * _DIM2) + b * _DIM2 + c
            j = k // (_CH // _L)
            idx_vs[j][pl.ds((k % (_CH // _L)) * _L, _L)] = idx

        gathers = [
            pltpu.async_copy(params_hbm.at[idx_vs[j]],
                             rows_v.at[pl.ds(j * _CH, _CH)], gsems[j])
            for j in range(_NCH)
        ]
        outs = []
        for j in range(_NCH):
            gathers[j].wait()
            outs.append(pltpu.async_copy(
                rows_v.at[pl.ds(j * _CH, _CH)],
                out_hbm.at[pl.ds(base + j * _CH, _CH)], osem))
        for o in outs:
            o.wait()

    return tabular_gather


_sc_call = _make_sc_call()


def kernel(state, params):
    flat = state.reshape(-1, state.shape[-1])
    state_t = flat.T.reshape(-1)
    return _sc_call(state_t, params)


# speedup vs baseline: 4.3986x; 4.3986x over previous
import functools

import jax
import jax.numpy as jnp
from jax import lax
from jax.experimental import pallas as pl
from jax.experimental.pallas import tpu as pltpu
from jax.experimental.pallas import tpu_sc as plsc

_NC = 2
_NS = 16
_NW = _NC * _NS
_L = 16

_B = 16384
_D = 64
_G = 8
_DIM0, _DIM1, _DIM2 = 100, 100, 100
_NSTATES = _DIM0 * _DIM1 * _DIM2
_BPW = _B // _NW


def _make_sc_call():
    mesh = plsc.VectorSubcoreMesh(core_axis_name="c", subcore_axis_name="s")

    @functools.partial(
        pl.kernel,
        mesh=mesh,
        out_type=jax.ShapeDtypeStruct((_G, _G, _B), jnp.float32),
        compiler_params=pltpu.CompilerParams(use_tc_tiling_on_sc=True,
                                             needs_layout_passes=False),
        scratch_types=[
            pltpu.VMEM((_BPW,), jnp.int32),
            pltpu.VMEM((_BPW,), jnp.int32),
            pltpu.VMEM((_BPW,), jnp.int32),
            pltpu.VMEM((_BPW,), jnp.int32),
            pltpu.VMEM((_G, _G, _G, 128), jnp.float32),
            pltpu.VMEM((_G, _G, _BPW), jnp.float32),
            pltpu.SemaphoreType.DMA,
            pltpu.SemaphoreType.DMA,
        ],
    )
    def tabular_gather(state_hbm, table_hbm, out_hbm,
                       s0_v, s1_v, s2_v, idx_v, blk_v, cols_v, gsem, osem):
        wid = lax.axis_index("s") * _NC + lax.axis_index("c")
        base = wid * _BPW

        pltpu.sync_copy(state_hbm.at[pl.ds(0 * _B + base, _BPW)], s0_v)
        pltpu.sync_copy(state_hbm.at[pl.ds(1 * _B + base, _BPW)], s1_v)
        pltpu.sync_copy(state_hbm.at[pl.ds(2 * _B + base, _BPW)], s2_v)

        for k in range(_BPW // _L):
            sl = pl.ds(k * _L, _L)
            a = jnp.minimum(jnp.maximum(s0_v[sl], 0), _DIM0 - 1)
            b = jnp.minimum(jnp.maximum(s1_v[sl], 0), _DIM1 - 1)
            c = jnp.minimum(jnp.maximum(s2_v[sl], 0), _DIM2 - 1)
            idx_v[sl] = a * (_DIM1 * _DIM2) + b * _DIM2 + c

        lanes = lax.iota(jnp.int32, _L)
        ghsel = []
        for q in range(_D // _L):
            j = lanes + jnp.int32(q * _L)
            ghsel.append((j >> 3, j & 7))

        @pl.loop(0, _BPW // _G)
        def _gather_group(k):
            vec = idx_v[pl.ds((k >> 1) * _L, _L)]
            jbase = (k & 1) * _G
            rs = []
            copies = []
            for j in range(_G):
                r = lax.reduce_max(
                    jnp.where(lanes == jbase + j, vec, 0), axes=(0,))
                rs.append(r)
                r16 = pl.multiple_of((r >> 4) << 4, _L)
                copies.append(pltpu.async_copy(
                    table_hbm.at[:, :, pl.ds(r16, _L)],
                    blk_v.at[j, :, :, pl.ds(0, _L)], gsem))
            for cp in copies:
                cp.wait()
            for j in range(_G):
                off = jnp.broadcast_to(rs[j] & 15, (_L,))
                slot = jnp.broadcast_to(jnp.int32(j), (_L,))
                i_bc = jnp.broadcast_to(k * _G + j, (_L,))
                for q in range(_D // _L):
                    gq, hq = ghsel[q]
                    vals = plsc.load_gather(blk_v, [slot, gq, hq, off])
                    plsc.store_scatter(cols_v, [gq, hq, i_bc], vals)

        outs = [
            pltpu.async_copy(cols_v.at[g], out_hbm.at[g, :, pl.ds(base, _BPW)],
                             osem)
            for g in range(_G)
        ]
        for o in outs:
            o.wait()

    return tabular_gather


_sc_call = _make_sc_call()


def kernel(state, params):
    flat = state.reshape(-1, state.shape[-1])
    state_t = flat.T.reshape(-1)
    table3 = params.T.reshape(_G, _G, _NSTATES)
    out3 = _sc_call(state_t, table3)
    return out3.reshape(_D, _B).T


# speedup vs baseline: 4.9480x; 1.1249x over previous
import functools

import jax
import jax.numpy as jnp
from jax import lax
from jax.experimental import pallas as pl
from jax.experimental.pallas import tpu as pltpu
from jax.experimental.pallas import tpu_sc as plsc

_NC = 2
_NS = 16
_NW = _NC * _NS
_L = 16

_B = 16384
_D = 64
_G = 8
_DIM0, _DIM1, _DIM2 = 100, 100, 100
_NSTATES = _DIM0 * _DIM1 * _DIM2
_BPW = _B // _NW
_NGRP = _BPW // _G


def _make_sc_call():
    mesh = plsc.VectorSubcoreMesh(core_axis_name="c", subcore_axis_name="s")

    @functools.partial(
        pl.kernel,
        mesh=mesh,
        out_type=jax.ShapeDtypeStruct((_G, _G, _B), jnp.float32),
        compiler_params=pltpu.CompilerParams(use_tc_tiling_on_sc=True,
                                             needs_layout_passes=False),
        scratch_types=[
            pltpu.VMEM((_BPW,), jnp.int32),
            pltpu.VMEM((_BPW,), jnp.int32),
            pltpu.VMEM((_BPW,), jnp.int32),
            pltpu.VMEM((_BPW,), jnp.int32),
            pltpu.VMEM((_G, _G, 128), jnp.float32),
            pltpu.VMEM((_G, _G, 128), jnp.float32),
            pltpu.VMEM((_G, _G, _BPW), jnp.float32),
            pltpu.SemaphoreType.DMA,
            pltpu.SemaphoreType.DMA,
            pltpu.SemaphoreType.DMA,
        ],
    )
    def tabular_gather(state_hbm, table_hbm, out_hbm,
                       s0_v, s1_v, s2_v, idx_v, bank_a, bank_b, cols_v,
                       sem_a, sem_b, osem):
        wid = lax.axis_index("s") * _NC + lax.axis_index("c")
        base = wid * _BPW

        pltpu.sync_copy(state_hbm.at[pl.ds(0 * _B + base, _BPW)], s0_v)
        pltpu.sync_copy(state_hbm.at[pl.ds(1 * _B + base, _BPW)], s1_v)
        pltpu.sync_copy(state_hbm.at[pl.ds(2 * _B + base, _BPW)], s2_v)

        for k in range(_BPW // _L):
            sl = pl.ds(k * _L, _L)
            a = jnp.minimum(jnp.maximum(s0_v[sl], 0), _DIM0 - 1)
            b = jnp.minimum(jnp.maximum(s1_v[sl], 0), _DIM1 - 1)
            c = jnp.minimum(jnp.maximum(s2_v[sl], 0), _DIM2 - 1)
            idx_v[sl] = a * (_DIM1 * _DIM2) + b * _DIM2 + c

        lanes = lax.iota(jnp.int32, _L)
        ghsel = []
        for q in range(_D // _L):
            j = lanes + jnp.int32(q * _L)
            ghsel.append((j >> 3, j & 7))

        def extract(k, j):
            vec = idx_v[pl.ds((k >> 1) * _L, _L)]
            jb = (k & 1) * _G
            return lax.reduce_max(jnp.where(lanes == jb + j, vec, 0),
                                  axes=(0,))

        def fetch(k, bank, sem):
            for j in range(_G):
                r = extract(k, j)
                r16 = pl.multiple_of((r >> 4) << 4, _L)
                pltpu.async_copy(
                    table_hbm.at[:, :, pl.ds(r16, _L)],
                    bank.at[:, :, pl.ds(j * _L, _L)], sem)

        def drain(bank, sem):
            for j in range(_G):
                pltpu.make_async_copy(
                    table_hbm.at[:, :, pl.ds(0, _L)],
                    bank.at[:, :, pl.ds(j * _L, _L)], sem).wait()

        def select(k, bank):
            for j in range(_G):
                r = extract(k, j)
                off = jnp.broadcast_to(j * _L + (r & 15), (_L,))
                i_bc = jnp.broadcast_to(k * _G + j, (_L,))
                for q in range(_D // _L):
                    gq, hq = ghsel[q]
                    vals = plsc.load_gather(bank, [gq, hq, off])
                    plsc.store_scatter(cols_v, [gq, hq, i_bc], vals)

        fetch(0, bank_a, sem_a)

        @pl.loop(0, _NGRP // 2 - 1)
        def _pipelined(t):
            k0 = 2 * t
            fetch(k0 + 1, bank_b, sem_b)
            drain(bank_a, sem_a)
            select(k0, bank_a)
            fetch(k0 + 2, bank_a, sem_a)
            drain(bank_b, sem_b)
            select(k0 + 1, bank_b)

        fetch(_NGRP - 1, bank_b, sem_b)
        drain(bank_a, sem_a)
        select(_NGRP - 2, bank_a)
        drain(bank_b, sem_b)
        select(_NGRP - 1, bank_b)

        outs = [
            pltpu.async_copy(cols_v.at[g], out_hbm.at[g, :, pl.ds(base, _BPW)],
                             osem)
            for g in range(_G)
        ]
        for o in outs:
            o.wait()

    return tabular_gather


_sc_call = _make_sc_call()


def kernel(state, params):
    flat = state.reshape(-1, state.shape[-1])
    state_t = flat.T.reshape(-1)
    table3 = params.T.reshape(_G, _G, _NSTATES)
    out3 = _sc_call(state_t, table3)
    return out3.reshape(_D, _B).T
